# baseline (speedup 1.0000x reference)
import functools

import jax
import jax.numpy as jnp
import numpy as np
from jax import lax
from jax.experimental import pallas as pl
from jax.experimental.pallas import tpu as pltpu
from jax.experimental.pallas import tpu_sc as plsc

N = 10000
E = 320000
C = 6
D = 128
H = 16

NC = 2
NS = 16
CPC = C // NC

RE = 80
NR = E // RE
RPT = NR // NS
G = 10
NBLK = RPT // G
ZROWS = CPC * N // NS
OROWS = N // NS

def _mm_body(x_ref, w_ref, o_ref):
    o_ref[...] = jnp.dot(x_ref[...], w_ref[0], preferred_element_type=jnp.float32)


def _project(x, W):
    return pl.pallas_call(
        _mm_body,
        grid=(C,),
        in_specs=[
            pl.BlockSpec((N, D), lambda c: (0, 0)),
            pl.BlockSpec((1, D, H), lambda c: (c, 0, 0)),
        ],
        out_specs=pl.BlockSpec((N, H), lambda c: (c, 0)),
        out_shape=jax.ShapeDtypeStruct((C * N, H), jnp.float32),
    )(x, W)


def _sc_body(h_hbm, gi_hbm, ew_hbm, bb_hbm, out_hbm,
             acc_sp, sd_a, w_a, rows_a, sd_b, w_b, rows_b,
             obuf, bbuf, isem, gsem_a, gsem_b, ssem_a, ssem_b):
    core = lax.axis_index("c")
    sub = lax.axis_index("s")

    def _zero_row(r, _):
        obuf[r] = jnp.zeros((H,), jnp.float32)
        return 0
    lax.fori_loop(0, OROWS, _zero_row, 0)
    for z in range(ZROWS // OROWS):
        pltpu.sync_copy(obuf, acc_sp.at[pl.ds(sub * ZROWS + z * OROWS, OROWS)])
    plsc.subcore_barrier()

    def _fire_gathers(sdbuf, rbuf, sem):
        for g in range(G):
            pltpu.async_copy(h_hbm.at[sdbuf.at[1, g]], rbuf.at[g], sem)

    def _drain_gathers(sdbuf, rbuf, sem):
        for g in range(G):
            pltpu.make_async_copy(h_hbm.at[sdbuf.at[1, g]], rbuf.at[g], sem).wait()

    def _fire_scatters(rbuf, sdbuf, sem):
        for g in range(G):
            pltpu.async_copy(rbuf.at[g], acc_sp.at[sdbuf.at[0, g]], sem, add=True)

    def _drain_scatters(rbuf, sdbuf, sem):
        for g in range(G):
            pltpu.make_async_copy(rbuf.at[g], acc_sp.at[sdbuf.at[0, g]], sem).wait()

    def _mult(rbuf, wbuf):
        def _m(g, _):
            for s in range(RE // H):
                wvec = wbuf[g, pl.ds(s * H, H)]
                for k in range(H):
                    l = s * H + k
                    rbuf[g, l] = rbuf[g, l] * wvec[jnp.full((H,), k, jnp.int32)]
            return 0
        lax.fori_loop(0, G, _m, 0)

    for j in range(CPC):
        ch = core * CPC + j
        base = sub * RPT

        def _advance(i, b, sdP, wP, rP, gsemP, ssemP,
                     sdQ, wQ, rQ, gsemQ, ssemQ, first, ch=ch, base=base):
            rowQ = base + (b + 1) * G
            _drain_gathers(sdP, rP, gsemP)
            if first:
                @pl.when(i > 0)
                def _():
                    _drain_scatters(rQ, sdQ, ssemQ)
            else:
                _drain_scatters(rQ, sdQ, ssemQ)
            c1 = pltpu.async_copy(gi_hbm.at[ch, :, pl.ds(rowQ, G)], sdQ, isem)
            c2 = pltpu.async_copy(ew_hbm.at[ch, pl.ds(rowQ, G)], wQ, isem)
            c1.wait(); c2.wait()
            _fire_gathers(sdQ, rQ, gsemQ)
            _mult(rP, wP)
            _fire_scatters(rP, sdP, ssemP)

        pltpu.sync_copy(gi_hbm.at[ch, :, pl.ds(base, G)], sd_a)
        pltpu.sync_copy(ew_hbm.at[ch, pl.ds(base, G)], w_a)
        _fire_gathers(sd_a, rows_a, gsem_a)

        def _pair(i, _):
            _advance(i, 2 * i, sd_a, w_a, rows_a, gsem_a, ssem_a,
                     sd_b, w_b, rows_b, gsem_b, ssem_b, True)
            _advance(i, 2 * i + 1, sd_b, w_b, rows_b, gsem_b, ssem_b,
                     sd_a, w_a, rows_a, gsem_a, ssem_a, False)
            return 0
        lax.fori_loop(0, (NBLK - 1) // 2, _pair, 0)

        _drain_gathers(sd_a, rows_a, gsem_a)
        _drain_scatters(rows_b, sd_b, ssem_b)
        _mult(rows_a, w_a)
        _fire_scatters(rows_a, sd_a, ssem_a)
        _drain_scatters(rows_a, sd_a, ssem_a)

    plsc.subcore_barrier()

    for j in range(CPC):
        ch = core * CPC + j
        pltpu.sync_copy(bb_hbm.at[ch], bbuf)
        bvec = bbuf[...]
        r0 = sub * OROWS
        pltpu.sync_copy(acc_sp.at[pl.ds(j * N + r0, OROWS)], obuf)

        def _bias_relu(r, _, bvec=bvec):
            obuf[r] = jnp.maximum(obuf[r] + bvec, 0.0)
            return 0
        lax.fori_loop(0, OROWS, _bias_relu, 0)
        pltpu.sync_copy(obuf, out_hbm.at[pl.ds(r0, OROWS), pl.ds(ch * H, H)])


_sc_call = pl.kernel(
    _sc_body,
    out_type=jax.ShapeDtypeStruct((N, C * H), jnp.float32),
    mesh=plsc.VectorSubcoreMesh(core_axis_name="c", subcore_axis_name="s"),
    compiler_params=pltpu.CompilerParams(use_tc_tiling_on_sc=False),
    scratch_types=[
        pltpu.VMEM_SHARED((CPC * N, H), jnp.float32),
        pltpu.VMEM((2, G, RE), jnp.int32),
        pltpu.VMEM((G, RE), jnp.float32),
        pltpu.VMEM((G, RE, H), jnp.float32),
        pltpu.VMEM((2, G, RE), jnp.int32),
        pltpu.VMEM((G, RE), jnp.float32),
        pltpu.VMEM((G, RE, H), jnp.float32),
        pltpu.VMEM((OROWS, H), jnp.float32),
        pltpu.VMEM((H,), jnp.float32),
        pltpu.SemaphoreType.DMA,
        pltpu.SemaphoreType.DMA,
        pltpu.SemaphoreType.DMA,
        pltpu.SemaphoreType.DMA,
        pltpu.SemaphoreType.DMA,
    ],
)


@jax.jit
def kernel(x, edge_index, edge_weight, W, b):
    h = _project(x, W)
    hoff = jnp.arange(C, dtype=jnp.int32) * N
    aoff = (jnp.arange(C, dtype=jnp.int32) % CPC) * N
    gi = jnp.stack(
        [edge_index[:, 0, :] + aoff[:, None],
         edge_index[:, 1, :] + hoff[:, None]],
        axis=1,
    ).reshape(C, 2, NR, RE)
    ew = edge_weight.reshape(C, NR, RE)
    bb = b.reshape(C, H)
    return _sc_call(h, gi, ew, bb)
